# baseline (speedup 1.0000x reference)
import functools

import jax
import jax.numpy as jnp
from jax import lax
from jax.experimental import pallas as pl
from jax.experimental.pallas import tpu as pltpu
from jax.experimental.pallas import tpu_sc as plsc

B, T = 16384, 50
V, D = 1000, 64
NC, NS = 2, 16
NW = NC * NS
B_PER_W = B // NW
RB = 8
N_CHUNKS = B_PER_W // RB
NBUF = 3

_mesh = plsc.VectorSubcoreMesh(
    core_axis_name="c", subcore_axis_name="s", num_cores=NC, num_subcores=NS
)


@functools.partial(
    pl.kernel,
    out_type=jax.ShapeDtypeStruct((B, T, D), jnp.float32),
    mesh=_mesh,
    compiler_params=pltpu.CompilerParams(use_tc_tiling_on_sc=False),
    scratch_types=[
        pltpu.VMEM((B_PER_W, T), jnp.int32),
        pltpu.VMEM((NBUF, RB, T, D), jnp.float32),
        pltpu.VMEM_SHARED((V, D), jnp.float32),
        pltpu.SemaphoreType.DMA((NBUF,)),
        pltpu.SemaphoreType.DMA((NBUF,)),
    ],
)
def _gather_kernel(
    x_hbm, table_hbm, out_hbm, idx_all, rows_v, table_sh, sem_g, sem_s
):
    wid = lax.axis_index("s") * NC + lax.axis_index("c")
    b0 = wid * B_PER_W

    @pl.when(lax.axis_index("s") == 0)
    def _():
        pltpu.sync_copy(table_hbm, table_sh)

    plsc.subcore_barrier()

    def issue_gather(i, b):
        for j in range(RB):
            pltpu.async_copy(
                table_sh.at[idx_all.at[i * RB + j]],
                rows_v.at[b, j],
                sem_g.at[b],
            )

    def wait_gather(b):
        pltpu.make_async_copy(
            out_hbm.at[pl.ds(0, RB)], rows_v.at[b], sem_g.at[b]
        ).wait()

    def out_slice(i):
        return out_hbm.at[pl.ds(b0 + i * RB, RB)]

    def issue_store(i, b):
        pltpu.async_copy(rows_v.at[b], out_slice(i), sem_s.at[b])

    def wait_store(i, b):
        pltpu.make_async_copy(rows_v.at[b], out_slice(i), sem_s.at[b]).wait()

    pltpu.sync_copy(x_hbm.at[pl.ds(b0, B_PER_W)], idx_all)

    issue_gather(0, 0)
    issue_gather(1, 1)
    wait_gather(0)
    issue_store(0, 0)
    issue_gather(2, 2)

    def body(i, carry):
        b = i % NBUF
        wait_gather(b)
        issue_store(i, b)
        bn = (i + 2) % NBUF
        wait_store(i - 1, bn)
        issue_gather(i + 2, bn)
        return carry

    lax.fori_loop(1, N_CHUNKS - 2, body, 0)

    wait_gather((N_CHUNKS - 2) % NBUF)
    issue_store(N_CHUNKS - 2, (N_CHUNKS - 2) % NBUF)
    wait_gather((N_CHUNKS - 1) % NBUF)
    issue_store(N_CHUNKS - 1, (N_CHUNKS - 1) % NBUF)
    for i in range(N_CHUNKS - 3, N_CHUNKS):
        wait_store(i, i % NBUF)


def kernel(x, table):
    return _gather_kernel(x, table)


# speedup vs baseline: 1.2797x; 1.2797x over previous
import functools

import jax
import jax.numpy as jnp
from jax import lax
from jax.experimental import pallas as pl
from jax.experimental.pallas import tpu as pltpu
from jax.experimental.pallas import tpu_sc as plsc

B, T = 16384, 50
V, D = 1000, 64
VP = 1024
NC, NS = 2, 16
NW = NC * NS
B_PER_W = B // NW
BH = 256
N_T = T

_mesh = plsc.VectorSubcoreMesh(
    core_axis_name="c", subcore_axis_name="s", num_cores=NC, num_subcores=NS
)


@functools.partial(
    pl.kernel,
    out_type=jax.ShapeDtypeStruct((T, D, B), jnp.float32),
    mesh=_mesh,
    compiler_params=pltpu.CompilerParams(needs_layout_passes=False),
    scratch_types=[
        pltpu.VMEM((B_PER_W * T,), jnp.int32),
        pltpu.VMEM((D, VP), jnp.float32),
        pltpu.VMEM((2, D, BH), jnp.float32),
        pltpu.SemaphoreType.DMA((2,)),
    ],
)
def _gather_kernel(x_hbm, tt_hbm, out_hbm, x_v, tt_v, st_v, sem_s):
    wid = lax.axis_index("s") * NC + lax.axis_index("c")
    b0 = wid * B_PER_W

    pltpu.sync_copy(x_hbm.at[pl.ds(b0 * T, B_PER_W * T)], x_v)
    pltpu.sync_copy(tt_hbm, tt_v)

    lane = lax.iota(jnp.int32, 16)
    pos50 = lane * T

    def fill(t, h):
        base = h * BH * T + t

        def blk_body(blk, carry):
            pos = pos50 + (base + blk * 16 * T)
            idx = plsc.load_gather(x_v, [pos])
            off = blk * 16
            for d in range(D):
                dvec = jnp.full((16,), d, jnp.int32)
                vals = plsc.load_gather(tt_v, [dvec, idx])
                st_v[h, d, pl.ds(off, 16)] = vals
            return carry

        lax.fori_loop(0, BH // 16, blk_body, 0)

    def out_slice(t, h):
        return out_hbm.at[t, :, pl.ds(b0 + h * BH, BH)]

    def issue_store(t, h):
        pltpu.async_copy(st_v.at[h], out_slice(t, h), sem_s.at[h])

    def wait_store(t, h):
        pltpu.make_async_copy(st_v.at[h], out_slice(t, h), sem_s.at[h]).wait()

    fill(0, 0)
    issue_store(0, 0)
    fill(0, 1)
    issue_store(0, 1)

    def body(t, carry):
        for h in range(2):
            wait_store(t - 1, h)
            fill(t, h)
            issue_store(t, h)
        return carry

    lax.fori_loop(1, N_T, body, 0)

    wait_store(N_T - 1, 0)
    wait_store(N_T - 1, 1)


def kernel(x, table):
    table_t = jnp.pad(table.T, ((0, 0), (0, VP - V)))
    out_t = _gather_kernel(x.reshape(-1), table_t)
    return out_t.transpose(2, 0, 1)


# speedup vs baseline: 1.4870x; 1.1620x over previous
import functools

import jax
import jax.numpy as jnp
from jax import lax
from jax.experimental import pallas as pl
from jax.experimental.pallas import tpu as pltpu
from jax.experimental.pallas import tpu_sc as plsc

B, T = 16384, 50
V, D = 1000, 64
VP = 1024
NC, NS = 2, 16
NW = NC * NS
B_PER_W = B // NW
BH = 256
N_T = T

_mesh = plsc.VectorSubcoreMesh(
    core_axis_name="c", subcore_axis_name="s", num_cores=NC, num_subcores=NS
)


@functools.partial(
    pl.kernel,
    out_type=jax.ShapeDtypeStruct((T, D, B), jnp.float32),
    mesh=_mesh,
    compiler_params=pltpu.CompilerParams(needs_layout_passes=False),
    scratch_types=[
        pltpu.VMEM((B_PER_W * T,), jnp.int32),
        pltpu.VMEM((D, VP), jnp.float32),
        pltpu.VMEM((2, D, BH), jnp.float32),
        pltpu.SemaphoreType.DMA((2,)),
    ],
)
def _gather_kernel(x_hbm, tt_hbm, out_hbm, x_v, tt_v, st_v, sem_s):
    wid = lax.axis_index("s") * NC + lax.axis_index("c")
    b0 = wid * B_PER_W

    pltpu.sync_copy(x_hbm.at[pl.ds(b0 * T, B_PER_W * T)], x_v)
    pltpu.sync_copy(tt_hbm, tt_v)

    lane = lax.iota(jnp.int32, 16)
    pos50 = lane * T

    def fill(t, h):
        base = h * BH * T + t

        @plsc.parallel_loop(0, BH // 16, unroll=2)
        def blk_body(blk):
            pos = pos50 + (base + blk * 16 * T)
            idx = plsc.load_gather(x_v, [pos])
            off = blk * 16
            for d0 in range(0, D, 8):
                vals = [
                    plsc.load_gather(
                        tt_v, [jnp.full((16,), d0 + k, jnp.int32), idx]
                    )
                    for k in range(8)
                ]
                for k in range(8):
                    st_v[h, d0 + k, pl.ds(off, 16)] = vals[k]

    def out_slice(t, h):
        return out_hbm.at[t, :, pl.ds(b0 + h * BH, BH)]

    def issue_store(t, h):
        pltpu.async_copy(st_v.at[h], out_slice(t, h), sem_s.at[h])

    def wait_store(t, h):
        pltpu.make_async_copy(st_v.at[h], out_slice(t, h), sem_s.at[h]).wait()

    fill(0, 0)
    issue_store(0, 0)
    fill(0, 1)
    issue_store(0, 1)

    def body(t, carry):
        for h in range(2):
            wait_store(t - 1, h)
            fill(t, h)
            issue_store(t, h)
        return carry

    lax.fori_loop(1, N_T, body, 0)

    wait_store(N_T - 1, 0)
    wait_store(N_T - 1, 1)


def kernel(x, table):
    table_t = jnp.pad(table.T, ((0, 0), (0, VP - V)))
    out_t = _gather_kernel(x.reshape(-1), table_t)
    return out_t.transpose(2, 0, 1)


# speedup vs baseline: 3.7656x; 2.5324x over previous
import functools

import jax
import jax.numpy as jnp
from jax import lax
from jax.experimental import pallas as pl
from jax.experimental.pallas import tpu as pltpu
from jax.experimental.pallas import tpu_sc as plsc

B, T = 16384, 50
V, D = 1000, 64
NC, NS = 2, 16
NW = NC * NS
B_PER_W = B // NW
BH = 256

_mesh = plsc.VectorSubcoreMesh(
    core_axis_name="c", subcore_axis_name="s", num_cores=NC, num_subcores=NS
)


@functools.partial(
    pl.kernel,
    out_type=jax.ShapeDtypeStruct((T, D, B), jnp.float32),
    mesh=_mesh,
    compiler_params=pltpu.CompilerParams(needs_layout_passes=False),
    scratch_types=[
        pltpu.VMEM((T, B_PER_W), jnp.int32),
        pltpu.VMEM((D, V), jnp.float32),
        pltpu.VMEM((2, D, BH), jnp.float32),
        pltpu.SemaphoreType.DMA((2,)),
    ],
)
def _gather_kernel(xt_hbm, tt_hbm, out_hbm, x_v, tt_v, st_v, sem_s):
    wid = lax.axis_index("s") * NC + lax.axis_index("c")
    b0 = wid * B_PER_W

    pltpu.sync_copy(xt_hbm.at[:, pl.ds(b0, B_PER_W)], x_v)
    pltpu.sync_copy(tt_hbm, tt_v)

    def fill(t, h):
        @plsc.parallel_loop(0, BH // 16, unroll=4)
        def blk_body(blk):
            off = blk * 16
            idx = x_v[t, pl.ds(h * BH + off, 16)]
            for d0 in range(0, D, 4):
                vals = [
                    plsc.load_gather(
                        tt_v, [jnp.full((16,), d0 + k, jnp.int32), idx]
                    )
                    for k in range(4)
                ]
                for k in range(4):
                    st_v[h, d0 + k, pl.ds(off, 16)] = vals[k]

    def out_slice(t, h):
        return out_hbm.at[t, :, pl.ds(b0 + h * BH, BH)]

    def issue_store(t, h):
        pltpu.async_copy(st_v.at[h], out_slice(t, h), sem_s.at[h])

    def wait_store(t, h):
        pltpu.make_async_copy(st_v.at[h], out_slice(t, h), sem_s.at[h]).wait()

    fill(0, 0)
    issue_store(0, 0)
    fill(0, 1)
    issue_store(0, 1)

    def body(t, carry):
        for h in range(2):
            wait_store(t - 1, h)
            fill(t, h)
            issue_store(t, h)
        return carry

    lax.fori_loop(1, T, body, 0)

    wait_store(T - 1, 0)
    wait_store(T - 1, 1)


def kernel(x, table):
    out_t = _gather_kernel(x.T, table.T)
    return out_t.transpose(2, 0, 1)
